# Initial kernel scaffold; baseline (speedup 1.0000x reference)
import jax
import jax.numpy as jnp
from jax.experimental import pallas as pl


def kernel(x, Wr, br, expert_up, expert_down):
    raise NotImplementedError("write your pallas kernel here")


# speedup vs baseline: 1.9281x; 1.9281x over previous
import functools

import jax
import jax.numpy as jnp
from jax import lax
from jax.experimental import pallas as pl
from jax.experimental.pallas import tpu as pltpu

_HIGH = lax.Precision.HIGHEST


def _router_kernel(x_ref, wr_ref, br_ref, idx_ref, wts_ref, nblk_ref, *, T, E, CAP, SUB):
    x = x_ref[...]
    logits = lax.dot_general(x, wr_ref[...], (((1,), (1,)), ((), ())),
                             preferred_element_type=jnp.float32)
    logits = logits + br_ref[...]
    m = jnp.max(logits, axis=1, keepdims=True)
    p = jnp.exp(logits - m)
    p = p / jnp.sum(p, axis=1, keepdims=True)

    ie = lax.broadcasted_iota(jnp.int32, (T, E), 1)
    m1 = jnp.max(p, axis=1, keepdims=True)
    am1 = jnp.min(jnp.where(p == m1, ie, E), axis=1, keepdims=True)
    pm = jnp.where(ie == am1, -1.0, p)
    m2 = jnp.max(pm, axis=1, keepdims=True)
    am2 = jnp.min(jnp.where(pm == m2, ie, E), axis=1, keepdims=True)

    oh1 = (ie == am1).astype(jnp.float32)
    oh2 = (ie == am2).astype(jnp.float32)
    both = oh1 + oh2

    it = lax.broadcasted_iota(jnp.int32, (T, T), 0)
    jt = lax.broadcasted_iota(jnp.int32, (T, T), 1)
    Ltri = (jt < it).astype(jnp.float32)
    cnt = lax.dot_general(Ltri, both, (((1,), (0,)), ((), ())),
                          preferred_element_type=jnp.float32, precision=_HIGH)
    r0 = jnp.sum(oh1 * cnt, axis=1, keepdims=True)
    r1 = jnp.sum(oh2 * cnt, axis=1, keepdims=True)

    ne = jnp.sum(both, axis=0, keepdims=True)
    nblk = jnp.ceil(ne * (1.0 / SUB)).astype(jnp.int32)
    nblk_ref[...] = nblk

    f0 = am1 * CAP + (r0 + 0.5).astype(jnp.int32)
    f1 = am2 * CAP + (r1 + 0.5).astype(jnp.int32)
    fi = lax.broadcasted_iota(jnp.int32, (T, E * CAP), 1)
    M0 = (fi == f0).astype(jnp.float32)
    M1 = (fi == f1).astype(jnp.float32)
    tcol = lax.broadcasted_iota(jnp.int32, (T, 1), 0).astype(jnp.float32)
    c00 = (((0,), (0,)), ((), ()))
    idx_flat = (lax.dot_general(M0, tcol, c00, preferred_element_type=jnp.float32, precision=_HIGH)
                + lax.dot_general(M1, tcol, c00, preferred_element_type=jnp.float32, precision=_HIGH))
    wts_flat = (lax.dot_general(M0, m1, c00, preferred_element_type=jnp.float32, precision=_HIGH)
                + lax.dot_general(M1, m2, c00, preferred_element_type=jnp.float32, precision=_HIGH))
    idx_ref[...] = (idx_flat + 0.5).astype(jnp.int32)
    wts_ref[...] = wts_flat


def _expert_kernel(nblk_ref, idx_ref, wts_ref, x_ref, up_ref, dn_ref, out_ref,
                   xg_ref, acc_ref, *, T, CAP, SUB, NH):
    e = pl.program_id(0)
    h = pl.program_id(1)
    nblk = nblk_ref[e]
    idx = idx_ref[0]
    itok = lax.broadcasted_iota(jnp.int32, (CAP, T), 1)
    G = (itok == idx).astype(jnp.float32)

    @pl.when(h == 0)
    def _():
        xg_ref[...] = lax.dot_general(G, x_ref[...], (((1,), (0,)), ((), ())),
                                      preferred_element_type=jnp.float32)
        acc_ref[...] = jnp.zeros_like(acc_ref)

    up = up_ref[0]
    dn = dn_ref[0]
    for sub in range(CAP // SUB):
        @pl.when(sub < nblk)
        def _():
            xs = xg_ref[sub * SUB:(sub + 1) * SUB, :]
            hp = lax.dot_general(xs, up, (((1,), (1,)), ((), ())),
                                 preferred_element_type=jnp.float32)
            g = hp * (0.5 * (1.0 + lax.erf(hp * 0.7071067811865476)))
            contrib = lax.dot_general(g, dn, (((1,), (1,)), ((), ())),
                                      preferred_element_type=jnp.float32)
            acc_ref[sub * SUB:(sub + 1) * SUB, :] += contrib

    @pl.when(h == NH - 1)
    def _():
        Gw = G * wts_ref[0]
        contribution = lax.dot_general(Gw, acc_ref[...], (((0,), (0,)), ((), ())),
                                       preferred_element_type=jnp.float32)

        @pl.when(e == 0)
        def _():
            out_ref[...] = contribution

        @pl.when(e > 0)
        def _():
            out_ref[...] += contribution


def kernel(x, Wr, br, expert_up, expert_down):
    Bsz, Ssz, D = x.shape
    E, H = expert_up.shape[0], expert_up.shape[1]
    T = Bsz * Ssz
    CAP = T
    SUB = 128
    HC = 768
    NH = H // HC
    xf = x.reshape(T, D)

    router = functools.partial(_router_kernel, T=T, E=E, CAP=CAP, SUB=SUB)
    idx_flat, wts_flat, nblk = pl.pallas_call(
        router,
        out_shape=[
            jax.ShapeDtypeStruct((E * CAP, 1), jnp.int32),
            jax.ShapeDtypeStruct((E * CAP, 1), jnp.float32),
            jax.ShapeDtypeStruct((1, E), jnp.int32),
        ],
    )(xf, Wr, br.reshape(1, E))

    expert = functools.partial(_expert_kernel, T=T, CAP=CAP, SUB=SUB, NH=NH)
    grid_spec = pltpu.PrefetchScalarGridSpec(
        num_scalar_prefetch=1,
        grid=(E, NH),
        in_specs=[
            pl.BlockSpec((1, CAP, 1), lambda e, h, s: (e, 0, 0)),
            pl.BlockSpec((1, CAP, 1), lambda e, h, s: (e, 0, 0)),
            pl.BlockSpec((T, D), lambda e, h, s: (0, 0)),
            pl.BlockSpec((1, HC, D), lambda e, h, s: (e, h, 0)),
            pl.BlockSpec((1, D, HC), lambda e, h, s: (e, 0, h)),
        ],
        out_specs=pl.BlockSpec((T, D), lambda e, h, s: (0, 0)),
        scratch_shapes=[
            pltpu.VMEM((CAP, D), jnp.float32),
            pltpu.VMEM((CAP, D), jnp.float32),
        ],
    )
    out = pl.pallas_call(
        expert,
        grid_spec=grid_spec,
        out_shape=jax.ShapeDtypeStruct((T, D), jnp.float32),
    )(nblk.reshape(E), idx_flat.reshape(E, CAP, 1), wts_flat.reshape(E, CAP, 1),
      xf, expert_up, expert_down)
    return out.reshape(Bsz, Ssz, D)


# speedup vs baseline: 2.4256x; 1.2580x over previous
import functools

import jax
import jax.numpy as jnp
from jax import lax
from jax.experimental import pallas as pl
from jax.experimental.pallas import tpu as pltpu

_HIGH = lax.Precision.HIGHEST


def _router_kernel(x_ref, wr_ref, br_ref, mall_ref, mw_ref, nblk_ref, *, T, E, CAP, SUB):
    x = x_ref[...]
    logits = lax.dot_general(x, wr_ref[...], (((1,), (1,)), ((), ())),
                             preferred_element_type=jnp.float32)
    logits = logits + br_ref[...]
    m = jnp.max(logits, axis=1, keepdims=True)
    p = jnp.exp(logits - m)
    p = p / jnp.sum(p, axis=1, keepdims=True)

    ie = lax.broadcasted_iota(jnp.int32, (T, E), 1)
    m1 = jnp.max(p, axis=1, keepdims=True)
    am1 = jnp.min(jnp.where(p == m1, ie, E), axis=1, keepdims=True)
    pm = jnp.where(ie == am1, -1.0, p)
    m2 = jnp.max(pm, axis=1, keepdims=True)
    am2 = jnp.min(jnp.where(pm == m2, ie, E), axis=1, keepdims=True)

    oh1 = (ie == am1).astype(jnp.float32)
    oh2 = (ie == am2).astype(jnp.float32)
    both = oh1 + oh2

    it = lax.broadcasted_iota(jnp.int32, (T, T), 0)
    jt = lax.broadcasted_iota(jnp.int32, (T, T), 1)
    Ltri = (jt < it).astype(jnp.float32)
    cnt = lax.dot_general(Ltri, both, (((1,), (0,)), ((), ())),
                          preferred_element_type=jnp.float32, precision=_HIGH)
    r0 = jnp.sum(oh1 * cnt, axis=1, keepdims=True)
    r1 = jnp.sum(oh2 * cnt, axis=1, keepdims=True)

    ne = jnp.sum(both, axis=0, keepdims=True)
    nblk = jnp.ceil(ne * (1.0 / SUB)).astype(jnp.int32)
    nblk_ref[...] = nblk

    f0 = am1 * CAP + (r0 + 0.5).astype(jnp.int32)
    f1 = am2 * CAP + (r1 + 0.5).astype(jnp.int32)
    fi = lax.broadcasted_iota(jnp.int32, (T, E * CAP), 1)
    M0 = (fi == f0).astype(jnp.float32)
    M1 = (fi == f1).astype(jnp.float32)
    mall_ref[...] = M0 + M1
    mw_ref[...] = M0 * m1 + M1 * m2


def _expert_kernel(nblk_ref, mall_ref, mw_ref, x_ref, up_ref, dn_ref, out_ref,
                   xg_ref, acc_ref, *, CAP, SUB, NH):
    e = pl.program_id(0)
    h = pl.program_id(1)
    nblk = nblk_ref[e]

    @pl.when(h == 0)
    def _():
        xg_ref[...] = lax.dot_general(mall_ref[...], x_ref[...],
                                      (((0,), (0,)), ((), ())),
                                      preferred_element_type=jnp.float32)
        acc_ref[...] = jnp.zeros_like(acc_ref)

    up = up_ref[0]
    dn = dn_ref[0]
    for sub in range(CAP // SUB):
        @pl.when(sub < nblk)
        def _():
            xs = xg_ref[sub * SUB:(sub + 1) * SUB, :]
            hp = lax.dot_general(xs, up, (((1,), (1,)), ((), ())),
                                 preferred_element_type=jnp.float32)
            g = hp * (0.5 * (1.0 + lax.erf(hp * 0.7071067811865476)))
            contrib = lax.dot_general(g, dn, (((1,), (1,)), ((), ())),
                                      preferred_element_type=jnp.float32)
            acc_ref[sub * SUB:(sub + 1) * SUB, :] += contrib

    @pl.when(h == NH - 1)
    def _():
        contribution = lax.dot_general(mw_ref[...], acc_ref[...],
                                       (((1,), (0,)), ((), ())),
                                       preferred_element_type=jnp.float32)

        @pl.when(e == 0)
        def _():
            out_ref[...] = contribution

        @pl.when(e > 0)
        def _():
            out_ref[...] += contribution


def kernel(x, Wr, br, expert_up, expert_down):
    Bsz, Ssz, D = x.shape
    E, H = expert_up.shape[0], expert_up.shape[1]
    T = Bsz * Ssz
    CAP = T
    SUB = 128
    HC = 1536
    NH = H // HC
    xf = x.reshape(T, D)

    router = functools.partial(_router_kernel, T=T, E=E, CAP=CAP, SUB=SUB)
    mall, mw, nblk = pl.pallas_call(
        router,
        out_shape=[
            jax.ShapeDtypeStruct((T, E * CAP), jnp.float32),
            jax.ShapeDtypeStruct((T, E * CAP), jnp.float32),
            jax.ShapeDtypeStruct((1, E), jnp.int32),
        ],
    )(xf, Wr, br.reshape(1, E))

    expert = functools.partial(_expert_kernel, CAP=CAP, SUB=SUB, NH=NH)
    grid_spec = pltpu.PrefetchScalarGridSpec(
        num_scalar_prefetch=1,
        grid=(E, NH),
        in_specs=[
            pl.BlockSpec((T, CAP), lambda e, h, s: (0, e)),
            pl.BlockSpec((T, CAP), lambda e, h, s: (0, e)),
            pl.BlockSpec((T, D), lambda e, h, s: (0, 0)),
            pl.BlockSpec((1, HC, D), lambda e, h, s: (e, h, 0)),
            pl.BlockSpec((1, D, HC), lambda e, h, s: (e, 0, h)),
        ],
        out_specs=pl.BlockSpec((T, D), lambda e, h, s: (0, 0)),
        scratch_shapes=[
            pltpu.VMEM((CAP, D), jnp.float32),
            pltpu.VMEM((CAP, D), jnp.float32),
        ],
    )
    out = pl.pallas_call(
        expert,
        grid_spec=grid_spec,
        out_shape=jax.ShapeDtypeStruct((T, D), jnp.float32),
    )(nblk.reshape(E), mall, mw, xf, expert_up, expert_down)
    return out.reshape(Bsz, Ssz, D)


# speedup vs baseline: 2.4919x; 1.0274x over previous
import functools

import jax
import jax.numpy as jnp
from jax import lax
from jax.experimental import pallas as pl
from jax.experimental.pallas import tpu as pltpu

_HIGH = lax.Precision.HIGHEST


def _router_kernel(x_ref, wr_ref, br_ref, mall_ref, mw_ref, nblk_ref, *, T, E, CAP, SUB):
    x = x_ref[...]
    logits = lax.dot_general(x, wr_ref[...], (((1,), (1,)), ((), ())),
                             preferred_element_type=jnp.float32)
    logits = logits + br_ref[...]
    m = jnp.max(logits, axis=1, keepdims=True)
    p = jnp.exp(logits - m)
    p = p / jnp.sum(p, axis=1, keepdims=True)

    ie = lax.broadcasted_iota(jnp.int32, (T, E), 1)
    m1 = jnp.max(p, axis=1, keepdims=True)
    am1 = jnp.min(jnp.where(p == m1, ie, E), axis=1, keepdims=True)
    pm = jnp.where(ie == am1, -1.0, p)
    m2 = jnp.max(pm, axis=1, keepdims=True)
    am2 = jnp.min(jnp.where(pm == m2, ie, E), axis=1, keepdims=True)

    oh1 = (ie == am1).astype(jnp.float32)
    oh2 = (ie == am2).astype(jnp.float32)
    both = oh1 + oh2

    it = lax.broadcasted_iota(jnp.int32, (T, T), 0)
    jt = lax.broadcasted_iota(jnp.int32, (T, T), 1)
    Ltri = (jt < it).astype(jnp.float32)
    cnt = lax.dot_general(Ltri, both, (((1,), (0,)), ((), ())),
                          preferred_element_type=jnp.float32, precision=_HIGH)
    r0 = jnp.sum(oh1 * cnt, axis=1, keepdims=True)
    r1 = jnp.sum(oh2 * cnt, axis=1, keepdims=True)

    ne = jnp.sum(both, axis=0, keepdims=True)
    nblk = jnp.ceil(ne * (1.0 / SUB)).astype(jnp.int32)
    nblk_ref[...] = nblk

    f0 = am1 * CAP + (r0 + 0.5).astype(jnp.int32)
    f1 = am2 * CAP + (r1 + 0.5).astype(jnp.int32)
    fi = lax.broadcasted_iota(jnp.int32, (T, E * CAP), 1)
    M0 = (fi == f0).astype(jnp.float32)
    M1 = (fi == f1).astype(jnp.float32)
    mall_ref[...] = M0 + M1
    mw_ref[...] = M0 * m1 + M1 * m2


def _expert_kernel(nblk_ref, mall_ref, mw_ref, x_ref, up_ref, dn_ref, out_ref,
                   xg_ref, acc_ref, *, CAP, SUB, NH):
    e = pl.program_id(0)
    h = pl.program_id(1)
    nblk = nblk_ref[e]

    @pl.when(h == 0)
    def _():
        xg_ref[...] = lax.dot_general(mall_ref[...], x_ref[...],
                                      (((0,), (0,)), ((), ())),
                                      preferred_element_type=jnp.float32)
        acc_ref[...] = jnp.zeros_like(acc_ref)

    up = up_ref[0].astype(jnp.bfloat16)
    dn = dn_ref[0].astype(jnp.bfloat16)
    for sub in range(CAP // SUB):
        @pl.when(sub < nblk)
        def _():
            xs = xg_ref[sub * SUB:(sub + 1) * SUB, :].astype(jnp.bfloat16)
            hp = lax.dot_general(xs, up, (((1,), (1,)), ((), ())),
                                 preferred_element_type=jnp.float32)
            g = hp * (0.5 * (1.0 + lax.erf(hp * 0.7071067811865476)))
            contrib = lax.dot_general(g.astype(jnp.bfloat16), dn,
                                      (((1,), (1,)), ((), ())),
                                      preferred_element_type=jnp.float32)
            acc_ref[sub * SUB:(sub + 1) * SUB, :] += contrib

    @pl.when(h == NH - 1)
    def _():
        contribution = lax.dot_general(mw_ref[...], acc_ref[...],
                                       (((1,), (0,)), ((), ())),
                                       preferred_element_type=jnp.float32)

        @pl.when(e == 0)
        def _():
            out_ref[...] = contribution

        @pl.when(e > 0)
        def _():
            out_ref[...] += contribution


def kernel(x, Wr, br, expert_up, expert_down):
    Bsz, Ssz, D = x.shape
    E, H = expert_up.shape[0], expert_up.shape[1]
    T = Bsz * Ssz
    CAP = T
    SUB = 128
    HC = 1536
    NH = H // HC
    xf = x.reshape(T, D)

    router = functools.partial(_router_kernel, T=T, E=E, CAP=CAP, SUB=SUB)
    mall, mw, nblk = pl.pallas_call(
        router,
        out_shape=[
            jax.ShapeDtypeStruct((T, E * CAP), jnp.float32),
            jax.ShapeDtypeStruct((T, E * CAP), jnp.float32),
            jax.ShapeDtypeStruct((1, E), jnp.int32),
        ],
    )(xf, Wr, br.reshape(1, E))

    expert = functools.partial(_expert_kernel, CAP=CAP, SUB=SUB, NH=NH)
    grid_spec = pltpu.PrefetchScalarGridSpec(
        num_scalar_prefetch=1,
        grid=(E, NH),
        in_specs=[
            pl.BlockSpec((T, CAP), lambda e, h, s: (0, e)),
            pl.BlockSpec((T, CAP), lambda e, h, s: (0, e)),
            pl.BlockSpec((T, D), lambda e, h, s: (0, 0)),
            pl.BlockSpec((1, HC, D), lambda e, h, s: (e, h, 0)),
            pl.BlockSpec((1, D, HC), lambda e, h, s: (e, 0, h)),
        ],
        out_specs=pl.BlockSpec((T, D), lambda e, h, s: (0, 0)),
        scratch_shapes=[
            pltpu.VMEM((CAP, D), jnp.float32),
            pltpu.VMEM((CAP, D), jnp.float32),
        ],
    )
    out = pl.pallas_call(
        expert,
        grid_spec=grid_spec,
        out_shape=jax.ShapeDtypeStruct((T, D), jnp.float32),
    )(nblk.reshape(E), mall, mw, xf, expert_up, expert_down)
    return out.reshape(Bsz, Ssz, D)


# speedup vs baseline: 2.5850x; 1.0373x over previous
import functools

import jax
import jax.numpy as jnp
from jax import lax
from jax.experimental import pallas as pl
from jax.experimental.pallas import tpu as pltpu

_HIGH = lax.Precision.HIGHEST


def _router_kernel(x_ref, wr_ref, br_ref, mall_ref, mw_ref, nblk_ref, *, T, E, CAP, SUB):
    x = x_ref[...]
    logits = lax.dot_general(x, wr_ref[...], (((1,), (1,)), ((), ())),
                             preferred_element_type=jnp.float32)
    logits = logits + br_ref[...]
    m = jnp.max(logits, axis=1, keepdims=True)
    p = jnp.exp(logits - m)
    p = p / jnp.sum(p, axis=1, keepdims=True)

    ie = lax.broadcasted_iota(jnp.int32, (T, E), 1)
    m1 = jnp.max(p, axis=1, keepdims=True)
    am1 = jnp.min(jnp.where(p == m1, ie, E), axis=1, keepdims=True)
    pm = jnp.where(ie == am1, -1.0, p)
    m2 = jnp.max(pm, axis=1, keepdims=True)
    am2 = jnp.min(jnp.where(pm == m2, ie, E), axis=1, keepdims=True)

    oh1 = (ie == am1).astype(jnp.float32)
    oh2 = (ie == am2).astype(jnp.float32)
    both = oh1 + oh2

    it = lax.broadcasted_iota(jnp.int32, (T, T), 0)
    jt = lax.broadcasted_iota(jnp.int32, (T, T), 1)
    Ltri = (jt < it).astype(jnp.float32)
    cnt = lax.dot_general(Ltri, both, (((1,), (0,)), ((), ())),
                          preferred_element_type=jnp.float32, precision=_HIGH)
    r0 = jnp.sum(oh1 * cnt, axis=1, keepdims=True)
    r1 = jnp.sum(oh2 * cnt, axis=1, keepdims=True)

    ne = jnp.sum(both, axis=0, keepdims=True)
    nblk = jnp.ceil(ne * (1.0 / SUB)).astype(jnp.int32)
    nblk_ref[...] = nblk

    f0 = am1 * CAP + (r0 + 0.5).astype(jnp.int32)
    f1 = am2 * CAP + (r1 + 0.5).astype(jnp.int32)
    fi = lax.broadcasted_iota(jnp.int32, (T, E * CAP), 1)
    M0 = (fi == f0).astype(jnp.float32)
    M1 = (fi == f1).astype(jnp.float32)
    mall_ref[...] = M0 + M1
    mw_ref[...] = M0 * m1 + M1 * m2


def _expert_kernel(nblk_ref, mall_ref, mw_ref, x_ref, up_ref, dn_ref, out_ref,
                   xg_ref, acc_ref, *, CAP, SUB, NH):
    e = pl.program_id(0)
    h = pl.program_id(1)
    nblk = nblk_ref[e]

    @pl.when(h == 0)
    def _():
        xg_ref[...] = lax.dot_general(mall_ref[...], x_ref[...],
                                      (((0,), (0,)), ((), ())),
                                      preferred_element_type=jnp.float32)
        acc_ref[...] = jnp.zeros_like(acc_ref)

    up = up_ref[0].astype(jnp.bfloat16)
    dn = dn_ref[0].astype(jnp.bfloat16)
    for sub in range(CAP // SUB):
        @pl.when(sub < nblk)
        def _():
            xs = xg_ref[sub * SUB:(sub + 1) * SUB, :].astype(jnp.bfloat16)
            hp = lax.dot_general(xs, up, (((1,), (1,)), ((), ())),
                                 preferred_element_type=jnp.float32)
            g = hp * (0.5 * (1.0 + lax.erf(hp * 0.7071067811865476)))
            contrib = lax.dot_general(g.astype(jnp.bfloat16), dn,
                                      (((1,), (1,)), ((), ())),
                                      preferred_element_type=jnp.float32)
            acc_ref[sub * SUB:(sub + 1) * SUB, :] += contrib

    @pl.when(h == NH - 1)
    def _():
        contribution = lax.dot_general(mw_ref[...], acc_ref[...],
                                       (((1,), (0,)), ((), ())),
                                       preferred_element_type=jnp.float32)

        @pl.when(e == 0)
        def _():
            out_ref[...] = contribution

        @pl.when(e > 0)
        def _():
            out_ref[...] += contribution


def kernel(x, Wr, br, expert_up, expert_down):
    Bsz, Ssz, D = x.shape
    E, H = expert_up.shape[0], expert_up.shape[1]
    T = Bsz * Ssz
    CAP = T
    SUB = 128
    HC = 3072
    NH = H // HC
    xf = x.reshape(T, D)

    router = functools.partial(_router_kernel, T=T, E=E, CAP=CAP, SUB=SUB)
    mall, mw, nblk = pl.pallas_call(
        router,
        out_shape=[
            jax.ShapeDtypeStruct((T, E * CAP), jnp.float32),
            jax.ShapeDtypeStruct((T, E * CAP), jnp.float32),
            jax.ShapeDtypeStruct((1, E), jnp.int32),
        ],
    )(xf, Wr, br.reshape(1, E))

    expert = functools.partial(_expert_kernel, CAP=CAP, SUB=SUB, NH=NH)
    grid_spec = pltpu.PrefetchScalarGridSpec(
        num_scalar_prefetch=1,
        grid=(E, NH),
        in_specs=[
            pl.BlockSpec((T, CAP), lambda e, h, s: (0, e)),
            pl.BlockSpec((T, CAP), lambda e, h, s: (0, e)),
            pl.BlockSpec((T, D), lambda e, h, s: (0, 0)),
            pl.BlockSpec((1, HC, D), lambda e, h, s: (e, h, 0)),
            pl.BlockSpec((1, D, HC), lambda e, h, s: (e, 0, h)),
        ],
        out_specs=pl.BlockSpec((T, D), lambda e, h, s: (0, 0)),
        scratch_shapes=[
            pltpu.VMEM((CAP, D), jnp.float32),
            pltpu.VMEM((CAP, D), jnp.float32),
        ],
    )
    out = pl.pallas_call(
        expert,
        grid_spec=grid_spec,
        out_shape=jax.ShapeDtypeStruct((T, D), jnp.float32),
    )(nblk.reshape(E), mall, mw, xf, expert_up, expert_down)
    return out.reshape(Bsz, Ssz, D)


# speedup vs baseline: 2.7153x; 1.0504x over previous
import functools

import jax
import jax.numpy as jnp
from jax import lax
from jax.experimental import pallas as pl
from jax.experimental.pallas import tpu as pltpu

_HIGH = lax.Precision.HIGHEST


def _router_kernel(x_ref, wr_ref, br_ref, mall_ref, mw_ref, nblk_ref, *, T, E, CAP, SUB):
    x = x_ref[...]
    logits = lax.dot_general(x, wr_ref[...], (((1,), (1,)), ((), ())),
                             preferred_element_type=jnp.float32)
    logits = logits + br_ref[...]
    m = jnp.max(logits, axis=1, keepdims=True)
    p = jnp.exp(logits - m)
    p = p / jnp.sum(p, axis=1, keepdims=True)

    ie = lax.broadcasted_iota(jnp.int32, (T, E), 1)
    m1 = jnp.max(p, axis=1, keepdims=True)
    am1 = jnp.min(jnp.where(p == m1, ie, E), axis=1, keepdims=True)
    pm = jnp.where(ie == am1, -1.0, p)
    m2 = jnp.max(pm, axis=1, keepdims=True)
    am2 = jnp.min(jnp.where(pm == m2, ie, E), axis=1, keepdims=True)

    oh1 = (ie == am1).astype(jnp.float32)
    oh2 = (ie == am2).astype(jnp.float32)
    both = oh1 + oh2

    it = lax.broadcasted_iota(jnp.int32, (T, T), 0)
    jt = lax.broadcasted_iota(jnp.int32, (T, T), 1)
    Ltri = (jt < it).astype(jnp.float32)
    cnt = lax.dot_general(Ltri, both, (((1,), (0,)), ((), ())),
                          preferred_element_type=jnp.float32, precision=_HIGH)
    r0 = jnp.sum(oh1 * cnt, axis=1, keepdims=True)
    r1 = jnp.sum(oh2 * cnt, axis=1, keepdims=True)

    ne = jnp.sum(both, axis=0, keepdims=True)
    nblk = jnp.ceil(ne * (1.0 / SUB)).astype(jnp.int32)
    nblk_ref[...] = nblk

    f0 = am1 * CAP + (r0 + 0.5).astype(jnp.int32)
    f1 = am2 * CAP + (r1 + 0.5).astype(jnp.int32)
    fi = lax.broadcasted_iota(jnp.int32, (T, E * CAP), 1)
    M0 = (fi == f0).astype(jnp.float32)
    M1 = (fi == f1).astype(jnp.float32)
    mall_ref[...] = M0 + M1
    mw_ref[...] = M0 * m1 + M1 * m2


def _expert_kernel(nblk_ref, mall_ref, mw_ref, x_ref, up_ref, dn_ref, out_ref,
                   xg_ref, *, CAP, SUB, HCH, NHC):
    e = pl.program_id(0)
    nblk = nblk_ref[e]

    @pl.when(e == 0)
    def _():
        out_ref[...] = jnp.zeros_like(out_ref)

    xg_ref[...] = lax.dot_general(mall_ref[...], x_ref[...],
                                  (((0,), (0,)), ((), ())),
                                  preferred_element_type=jnp.float32)

    for sub in range(CAP // SUB):
        @pl.when(sub < nblk)
        def _():
            xs = xg_ref[sub * SUB:(sub + 1) * SUB, :]
            contrib = None
            for c in range(NHC):
                up_c = up_ref[0, c * HCH:(c + 1) * HCH, :]
                dn_c = dn_ref[0, :, c * HCH:(c + 1) * HCH]
                hp = lax.dot_general(xs, up_c, (((1,), (1,)), ((), ())),
                                     preferred_element_type=jnp.float32)
                g = hp * (0.5 * (1.0 + lax.erf(hp * 0.7071067811865476)))
                d = lax.dot_general(g, dn_c, (((1,), (1,)), ((), ())),
                                    preferred_element_type=jnp.float32)
                contrib = d if contrib is None else contrib + d
            mws = mw_ref[:, sub * SUB:(sub + 1) * SUB]
            out_ref[...] += lax.dot_general(mws, contrib, (((1,), (0,)), ((), ())),
                                            preferred_element_type=jnp.float32)


def kernel(x, Wr, br, expert_up, expert_down):
    Bsz, Ssz, D = x.shape
    E, H = expert_up.shape[0], expert_up.shape[1]
    T = Bsz * Ssz
    CAP = T
    SUB = 128
    HCH = 1536
    NHC = H // HCH
    xf = x.reshape(T, D)

    router = functools.partial(_router_kernel, T=T, E=E, CAP=CAP, SUB=SUB)
    mall, mw, nblk = pl.pallas_call(
        router,
        out_shape=[
            jax.ShapeDtypeStruct((T, E * CAP), jnp.float32),
            jax.ShapeDtypeStruct((T, E * CAP), jnp.float32),
            jax.ShapeDtypeStruct((1, E), jnp.int32),
        ],
    )(xf, Wr, br.reshape(1, E))

    expert = functools.partial(_expert_kernel, CAP=CAP, SUB=SUB, HCH=HCH, NHC=NHC)
    grid_spec = pltpu.PrefetchScalarGridSpec(
        num_scalar_prefetch=1,
        grid=(E,),
        in_specs=[
            pl.BlockSpec((T, CAP), lambda e, s: (0, e)),
            pl.BlockSpec((T, CAP), lambda e, s: (0, e)),
            pl.BlockSpec((T, D), lambda e, s: (0, 0)),
            pl.BlockSpec((1, H, D), lambda e, s: (e, 0, 0)),
            pl.BlockSpec((1, D, H), lambda e, s: (e, 0, 0)),
        ],
        out_specs=pl.BlockSpec((T, D), lambda e, s: (0, 0)),
        scratch_shapes=[
            pltpu.VMEM((CAP, D), jnp.float32),
        ],
    )
    out = pl.pallas_call(
        expert,
        grid_spec=grid_spec,
        out_shape=jax.ShapeDtypeStruct((T, D), jnp.float32),
    )(nblk.reshape(E), mall, mw, xf, expert_up, expert_down)
    return out.reshape(Bsz, Ssz, D)


# speedup vs baseline: 2.8628x; 1.0543x over previous
import functools

import jax
import jax.numpy as jnp
from jax import lax
from jax.experimental import pallas as pl
from jax.experimental.pallas import tpu as pltpu

_HIGH = lax.Precision.HIGHEST


def _moe_kernel(x_ref, wr_ref, br_ref, up_ref, dn_ref, out_ref,
                mall_s, mw_s, xg_ref, nblk_s, *, T, E, CAP, SUB, HCH, NHC):
    e = pl.program_id(0)

    @pl.when(e == 0)
    def _():
        x = x_ref[...]
        logits = lax.dot_general(x, wr_ref[...], (((1,), (1,)), ((), ())),
                                 preferred_element_type=jnp.float32)
        logits = logits + br_ref[...]
        m = jnp.max(logits, axis=1, keepdims=True)
        p = jnp.exp(logits - m)
        p = p / jnp.sum(p, axis=1, keepdims=True)

        ie = lax.broadcasted_iota(jnp.int32, (T, E), 1)
        m1 = jnp.max(p, axis=1, keepdims=True)
        am1 = jnp.min(jnp.where(p == m1, ie, E), axis=1, keepdims=True)
        pm = jnp.where(ie == am1, -1.0, p)
        m2 = jnp.max(pm, axis=1, keepdims=True)
        am2 = jnp.min(jnp.where(pm == m2, ie, E), axis=1, keepdims=True)

        oh1 = (ie == am1).astype(jnp.float32)
        oh2 = (ie == am2).astype(jnp.float32)
        both = oh1 + oh2

        it = lax.broadcasted_iota(jnp.int32, (T, T), 0)
        jt = lax.broadcasted_iota(jnp.int32, (T, T), 1)
        Ltri = (jt < it).astype(jnp.float32)
        cnt = lax.dot_general(Ltri, both, (((1,), (0,)), ((), ())),
                              preferred_element_type=jnp.float32, precision=_HIGH)
        r0 = jnp.sum(oh1 * cnt, axis=1, keepdims=True)
        r1 = jnp.sum(oh2 * cnt, axis=1, keepdims=True)

        ne = jnp.sum(both, axis=0, keepdims=True)
        nblk = jnp.ceil(ne * (1.0 / SUB)).astype(jnp.int32)
        for ee in range(E):
            nblk_s[0, ee] = nblk[0, ee]

        f0 = am1 * CAP + (r0 + 0.5).astype(jnp.int32)
        f1 = am2 * CAP + (r1 + 0.5).astype(jnp.int32)
        fcap = lax.broadcasted_iota(jnp.int32, (T, CAP), 1)
        for ee in range(E):
            fi = fcap + ee * CAP
            M0e = (fi == f0).astype(jnp.float32)
            M1e = (fi == f1).astype(jnp.float32)
            mall_s[ee] = M0e + M1e
            mw_s[ee] = M0e * m1 + M1e * m2

        out_ref[...] = jnp.zeros_like(out_ref)

    nblk = nblk_s[0, e]
    xg_ref[...] = lax.dot_general(mall_s[e], x_ref[...],
                                  (((0,), (0,)), ((), ())),
                                  preferred_element_type=jnp.float32)

    for sub in range(CAP // SUB):
        @pl.when(sub < nblk)
        def _():
            xs = xg_ref[sub * SUB:(sub + 1) * SUB, :]
            contrib = None
            for c in range(NHC):
                up_c = up_ref[0, c * HCH:(c + 1) * HCH, :]
                dn_c = dn_ref[0, :, c * HCH:(c + 1) * HCH]
                hp = lax.dot_general(xs, up_c, (((1,), (1,)), ((), ())),
                                     preferred_element_type=jnp.float32)
                g = hp * (0.5 * (1.0 + lax.erf(hp * 0.7071067811865476)))
                d = lax.dot_general(g, dn_c, (((1,), (1,)), ((), ())),
                                    preferred_element_type=jnp.float32)
                contrib = d if contrib is None else contrib + d
            mws = mw_s[e][:, sub * SUB:(sub + 1) * SUB]
            out_ref[...] += lax.dot_general(mws, contrib, (((1,), (0,)), ((), ())),
                                            preferred_element_type=jnp.float32)


def kernel(x, Wr, br, expert_up, expert_down):
    Bsz, Ssz, D = x.shape
    E, H = expert_up.shape[0], expert_up.shape[1]
    T = Bsz * Ssz
    CAP = T
    SUB = 128
    HCH = 1536
    NHC = H // HCH
    xf = x.reshape(T, D)

    body = functools.partial(_moe_kernel, T=T, E=E, CAP=CAP, SUB=SUB,
                             HCH=HCH, NHC=NHC)
    out = pl.pallas_call(
        body,
        grid=(E,),
        in_specs=[
            pl.BlockSpec((T, D), lambda e: (0, 0)),
            pl.BlockSpec((E, D), lambda e: (0, 0)),
            pl.BlockSpec((1, E), lambda e: (0, 0)),
            pl.BlockSpec((1, H, D), lambda e: (e, 0, 0)),
            pl.BlockSpec((1, D, H), lambda e: (e, 0, 0)),
        ],
        out_specs=pl.BlockSpec((T, D), lambda e: (0, 0)),
        scratch_shapes=[
            pltpu.VMEM((E, T, CAP), jnp.float32),
            pltpu.VMEM((E, T, CAP), jnp.float32),
            pltpu.VMEM((CAP, D), jnp.float32),
            pltpu.SMEM((1, E), jnp.int32),
        ],
        out_shape=jax.ShapeDtypeStruct((T, D), jnp.float32),
    )(xf, Wr, br.reshape(1, E), expert_up, expert_down)
    return out.reshape(Bsz, Ssz, D)
